# Initial kernel scaffold; baseline (speedup 1.0000x reference)
import jax
import jax.numpy as jnp
from jax.experimental import pallas as pl


def kernel(x, embed, W1, b1, W2, b2):
    raise NotImplementedError("write your pallas kernel here")


# speedup vs baseline: 8.6417x; 8.6417x over previous
import functools

import jax
import jax.numpy as jnp
from jax import lax
from jax.experimental import pallas as pl
from jax.experimental.pallas import tpu as pltpu
from jax.experimental.pallas import tpu_sc as plsc

B = 16384
L = 50
D = 128
A = 1000
H = 64
AP = 1024
R = L * AP
AOUT = 1024

NC = 2
NS = 16
NW = NC * NS
SPW = B // NW
CHUNK = 16
ROWS = CHUNK * L
NCHUNKS = SPW // CHUNK


def _table_body(e_ref, w_ref, t_ref):
    t_ref[0] = jnp.dot(e_ref[:], w_ref[0], preferred_element_type=jnp.float32)


def _build_table(embed_p, w1t):
    return pl.pallas_call(
        _table_body,
        grid=(L,),
        in_specs=[
            pl.BlockSpec((AP, D), lambda l: (0, 0)),
            pl.BlockSpec((1, D, H), lambda l: (l, 0, 0)),
        ],
        out_specs=pl.BlockSpec((1, AP, H), lambda l: (l, 0, 0)),
        out_shape=jax.ShapeDtypeStruct((L, AP, H), jnp.float32),
    )(embed_p, w1t)


def _bag_body(table_hbm, idx_hbm, out_hbm, idx_v, rows_v, h1_v, sem):
    wid = lax.axis_index("s") * NC + lax.axis_index("c")
    base = wid * SPW

    def chunk_body(c, carry):
        s0 = base + c * CHUNK
        pltpu.sync_copy(idx_hbm.at[pl.ds(s0 * L, ROWS)], idx_v)
        pltpu.async_copy(table_hbm.at[idx_v], rows_v, sem).wait()

        def samp_body(s, carry2):
            r0 = s * L
            for j in range(H // 16):
                acc = rows_v[r0, pl.ds(j * 16, 16)]
                for l in range(1, L):
                    acc = acc + rows_v[r0 + l, pl.ds(j * 16, 16)]
                h1_v[s, pl.ds(j * 16, 16)] = acc
            return carry2

        lax.fori_loop(0, CHUNK, samp_body, 0)
        pltpu.sync_copy(h1_v, out_hbm.at[pl.ds(s0, CHUNK)])
        return carry

    lax.fori_loop(0, NCHUNKS, chunk_body, 0)


@functools.cache
def _bag():
    return pl.kernel(
        _bag_body,
        mesh=plsc.VectorSubcoreMesh(core_axis_name="c", subcore_axis_name="s"),
        compiler_params=pltpu.CompilerParams(use_tc_tiling_on_sc=False),
        out_type=jax.ShapeDtypeStruct((B, H), jnp.float32),
        scratch_types=[
            pltpu.VMEM((ROWS,), jnp.int32),
            pltpu.VMEM((ROWS, H), jnp.float32),
            pltpu.VMEM((CHUNK, H), jnp.float32),
            pltpu.SemaphoreType.DMA,
        ],
    )


BT = 2048


def _mlp2_body(h_ref, b1_ref, w2_ref, b2_ref, o_ref):
    h = jnp.maximum(h_ref[:] + b1_ref[:], 0.0)
    o_ref[:] = jnp.dot(h, w2_ref[:], preferred_element_type=jnp.float32) + b2_ref[:]


def _mlp2(h1, b1r, w2p, b2p):
    return pl.pallas_call(
        _mlp2_body,
        grid=(B // BT,),
        in_specs=[
            pl.BlockSpec((BT, H), lambda i: (i, 0)),
            pl.BlockSpec((1, H), lambda i: (0, 0)),
            pl.BlockSpec((H, AOUT), lambda i: (0, 0)),
            pl.BlockSpec((1, AOUT), lambda i: (0, 0)),
        ],
        out_specs=pl.BlockSpec((BT, AOUT), lambda i: (i, 0)),
        out_shape=jax.ShapeDtypeStruct((B, AOUT), jnp.float32),
    )(h1, b1r, w2p, b2p)


def kernel(x, embed, W1, b1, W2, b2):
    embed_p = jnp.zeros((AP, D), jnp.float32).at[: A + 1, :].set(embed)
    w1t = W1.reshape(H, L, D).transpose(1, 2, 0)
    table = _build_table(embed_p, w1t).reshape(R, H)

    flat_idx = (
        x.astype(jnp.int32) + (jnp.arange(L, dtype=jnp.int32) * AP)[None, :]
    ).reshape(-1)

    h1 = _bag()(table, flat_idx)

    w2p = jnp.zeros((H, AOUT), jnp.float32).at[:, :A].set(W2.T)
    b2p = jnp.zeros((1, AOUT), jnp.float32).at[0, :A].set(b2)
    out = _mlp2(h1, b1.reshape(1, H), w2p, b2p)
    return out[:, :A]


# speedup vs baseline: 10.7826x; 1.2477x over previous
import functools

import jax
import jax.numpy as jnp
from jax import lax
from jax.experimental import pallas as pl
from jax.experimental.pallas import tpu as pltpu
from jax.experimental.pallas import tpu_sc as plsc

B = 16384
L = 50
D = 128
A = 1000
H = 64
AP = 1024
R = L * AP
AOUT = 1024

NC = 2
NS = 16
NW = NC * NS
SPW = B // NW
CHUNK = 16
ROWS = CHUNK * L
NCHUNKS = SPW // CHUNK


def _table_body(e_ref, w_ref, t_ref):
    t_ref[0] = jnp.dot(e_ref[:], w_ref[0], preferred_element_type=jnp.float32)


def _build_table(embed_p, w1t):
    return pl.pallas_call(
        _table_body,
        grid=(L,),
        in_specs=[
            pl.BlockSpec((AP, D), lambda l: (0, 0)),
            pl.BlockSpec((1, D, H), lambda l: (l, 0, 0)),
        ],
        out_specs=pl.BlockSpec((1, AP, H), lambda l: (l, 0, 0)),
        out_shape=jax.ShapeDtypeStruct((L, AP, H), jnp.float32),
    )(embed_p, w1t)


def _bag_body(table_hbm, idx_hbm, out_hbm, idx_v, rows0, rows1, h1_v, sem0, sem1):
    wid = lax.axis_index("s") * NC + lax.axis_index("c")
    base = wid * SPW
    pltpu.sync_copy(idx_hbm.at[pl.ds(base * L, SPW * L)], idx_v)

    def gather(c, rows, sem):
        return pltpu.make_async_copy(
            table_hbm.at[idx_v.at[pl.ds(c * ROWS, ROWS)]], rows, sem
        )

    def accum(c, rows):
        def samp_body(s, carry):
            r0 = s * L
            for j in range(H // 16):
                acc = rows[r0, pl.ds(j * 16, 16)]
                for l in range(1, L):
                    acc = acc + rows[r0 + l, pl.ds(j * 16, 16)]
                h1_v[s, pl.ds(j * 16, 16)] = acc
            return carry

        lax.fori_loop(0, CHUNK, samp_body, 0)
        pltpu.sync_copy(h1_v, out_hbm.at[pl.ds(base + c * CHUNK, CHUNK)])

    last = NCHUNKS - 1
    gather(0, rows0, sem0).start()

    def pair_body(k, carry):
        c0 = 2 * k
        gather(c0, rows0, sem0).wait()
        gather(c0 + 1, rows1, sem1).start()
        accum(c0, rows0)
        gather(c0 + 1, rows1, sem1).wait()
        gather(lax.min(c0 + 2, last), rows0, sem0).start()
        accum(c0 + 1, rows1)
        return carry

    lax.fori_loop(0, NCHUNKS // 2, pair_body, 0)
    gather(last, rows0, sem0).wait()


@functools.cache
def _bag():
    return pl.kernel(
        _bag_body,
        mesh=plsc.VectorSubcoreMesh(core_axis_name="c", subcore_axis_name="s"),
        compiler_params=pltpu.CompilerParams(use_tc_tiling_on_sc=False),
        out_type=jax.ShapeDtypeStruct((B, H), jnp.float32),
        scratch_types=[
            pltpu.VMEM((SPW * L,), jnp.int32),
            pltpu.VMEM((ROWS, H), jnp.float32),
            pltpu.VMEM((ROWS, H), jnp.float32),
            pltpu.VMEM((CHUNK, H), jnp.float32),
            pltpu.SemaphoreType.DMA,
            pltpu.SemaphoreType.DMA,
        ],
    )


BT = 2048


def _mlp2_body(h_ref, b1_ref, w2_ref, b2_ref, o_ref):
    h = jnp.maximum(h_ref[:] + b1_ref[:], 0.0)
    o = jnp.dot(h, w2_ref[:], preferred_element_type=jnp.float32) + b2_ref[:]
    o_ref[:] = o[:, :A]


def _mlp2(h1, b1r, w2p, b2p):
    return pl.pallas_call(
        _mlp2_body,
        grid=(B // BT,),
        in_specs=[
            pl.BlockSpec((BT, H), lambda i: (i, 0)),
            pl.BlockSpec((1, H), lambda i: (0, 0)),
            pl.BlockSpec((H, AOUT), lambda i: (0, 0)),
            pl.BlockSpec((1, AOUT), lambda i: (0, 0)),
        ],
        out_specs=pl.BlockSpec((BT, A), lambda i: (i, 0)),
        out_shape=jax.ShapeDtypeStruct((B, A), jnp.float32),
    )(h1, b1r, w2p, b2p)


def kernel(x, embed, W1, b1, W2, b2):
    embed_p = jnp.zeros((AP, D), jnp.float32).at[: A + 1, :].set(embed)
    w1t = W1.reshape(H, L, D).transpose(1, 2, 0)
    table = _build_table(embed_p, w1t).reshape(R, H)

    flat_idx = (
        x.astype(jnp.int32) + (jnp.arange(L, dtype=jnp.int32) * AP)[None, :]
    ).reshape(-1)

    h1 = _bag()(table, flat_idx)

    w2p = jnp.zeros((H, AOUT), jnp.float32).at[:, :A].set(W2.T)
    b2p = jnp.zeros((1, AOUT), jnp.float32).at[0, :A].set(b2)
    return _mlp2(h1, b1.reshape(1, H), w2p, b2p)
